# baseline (speedup 1.0000x reference)
import functools

import jax
import jax.numpy as jnp
from jax import lax
from jax.experimental import pallas as pl
from jax.experimental.pallas import tpu as pltpu
from jax.experimental.pallas import tpu_sc as plsc

NC = 2
NS = 16
NW = NC * NS
LANES = 16

BLK = 128


def _make_gather(BATCH, HIST, D, VP):
    b_per_w = BATCH // NW
    n_blk = b_per_w // BLK
    n_step = HIST * n_blk
    mesh = plsc.VectorSubcoreMesh(
        core_axis_name="c", subcore_axis_name="s", num_cores=NC, num_subcores=NS)

    @functools.partial(
        pl.kernel,
        out_type=jax.ShapeDtypeStruct((HIST, D, BATCH), jnp.float32),
        mesh=mesh,
        scratch_types=[
            pltpu.VMEM((BLK,), jnp.int32),
            pltpu.VMEM((BLK,), jnp.int32),
            pltpu.VMEM((BLK,), jnp.int32),
            pltpu.VMEM((BLK,), jnp.int32),
            pltpu.VMEM((BLK,), jnp.int32),
            pltpu.VMEM((BLK,), jnp.int32),
            pltpu.VMEM((BLK, 128), jnp.float32),
            pltpu.VMEM((BLK, 128), jnp.float32),
            pltpu.VMEM((D, BLK), jnp.float32),
            pltpu.VMEM((D, BLK), jnp.float32),
            pltpu.SemaphoreType.DMA,
            pltpu.SemaphoreType.DMA,
            pltpu.SemaphoreType.DMA,
            pltpu.SemaphoreType.DMA,
            pltpu.SemaphoreType.DMA,
            pltpu.SemaphoreType.DMA,
        ],
        compiler_params=pltpu.CompilerParams(needs_layout_passes=False),
    )
    def gather_kernel(idx_hbm, packed_hbm, out_hbm,
                      ia, ib, gia, gib, qa, qb, ra, rb, oa, ob,
                      isa, isb, gsa, gsb, osa, osb):
        wid = lax.axis_index("s") * NC + lax.axis_index("c")
        base = wid * b_per_w
        iota = lax.iota(jnp.int32, LANES)

        def fire_idx(i, t, sem):
            h = t >> 2
            j = t & 3
            return pltpu.async_copy(
                idx_hbm.at[pl.ds(h * BATCH + base + j * BLK, BLK)], i, sem)

        def wait_idx(i, sem):
            pltpu.make_async_copy(idx_hbm.at[pl.ds(0, BLK)], i, sem).wait()

        def build(i, gi, q):
            for k in range(BLK // LANES):
                v = i[pl.ds(k * LANES, LANES)]
                gi[pl.ds(k * LANES, LANES)] = lax.shift_right_logical(v, 2)
                q[pl.ds(k * LANES, LANES)] = lax.shift_left(v & 3, 5)

        def fire_gather(gi, r, sem):
            return pltpu.async_copy(packed_hbm.at[gi], r, sem)

        def wait_gather(gi, r, sem):
            pltpu.make_async_copy(packed_hbm.at[gi], r, sem).wait()

        def compact(r, q, o):
            qvs = [q[pl.ds(b0 * LANES, LANES)] for b0 in range(BLK // LANES)]
            rows = [b0 * LANES + iota for b0 in range(BLK // LANES)]
            for d in range(D):
                for b0 in range(BLK // LANES):
                    o[d, pl.ds(b0 * LANES, LANES)] = plsc.load_gather(
                        r, [rows[b0], qvs[b0] + d])

        def store(o, t, sem):
            h = t >> 2
            j = t & 3
            return pltpu.async_copy(
                o, out_hbm.at[h, :, pl.ds(base + j * BLK, BLK)], sem)

        def wait_store(o, sem):
            pltpu.make_async_copy(o, out_hbm.at[0, :, pl.ds(0, BLK)], sem).wait()

        fire_idx(ia, 0, isa)
        wait_idx(ia, isa)
        build(ia, gia, qa)
        fire_gather(gia, ra, gsa)
        fire_idx(ib, 1, isb)

        def body(s):
            t0 = 2 * s

            @pl.when(s > 0)
            def _():
                wait_store(oa, osa)
            wait_gather(gia, ra, gsa)
            compact(ra, qa, oa)
            store(oa, t0, osa)

            wait_idx(ib, isb)
            build(ib, gib, qb)
            fire_gather(gib, rb, gsb)

            @pl.when(t0 + 2 < n_step)
            def _():
                fire_idx(ia, t0 + 2, isa)

            @pl.when(s > 0)
            def _():
                wait_store(ob, osb)
            wait_gather(gib, rb, gsb)
            compact(rb, qb, ob)
            store(ob, t0 + 1, osb)

            @pl.when(t0 + 2 < n_step)
            def _():
                wait_idx(ia, isa)
                build(ia, gia, qa)
                fire_gather(gia, ra, gsa)

            @pl.when(t0 + 3 < n_step)
            def _():
                fire_idx(ib, t0 + 3, isb)

        pl.loop(0, n_step // 2)(body)
        wait_store(oa, osa)
        wait_store(ob, osb)

    return gather_kernel


def kernel(inputs, table):
    BATCH, HIST = inputs.shape
    V, D = table.shape
    packed = jnp.reshape(table, (V * D // 128, 128))
    idx1d = jnp.reshape(jnp.transpose(inputs.astype(jnp.int32)), (BATCH * HIST,))
    outT = _make_gather(BATCH, HIST, D, V * D // 128)(idx1d, packed)
    return jnp.transpose(outT, (2, 0, 1))


# speedup vs baseline: 1.5519x; 1.5519x over previous
import functools

import jax
import jax.numpy as jnp
from jax import lax
from jax.experimental import pallas as pl
from jax.experimental.pallas import tpu as pltpu
from jax.experimental.pallas import tpu_sc as plsc

NC = 2
NS = 16
NW = NC * NS
LANES = 16

BLK = 128


def _make_gather(BATCH, HIST, D, VP):
    b_per_w = BATCH // NW
    n_blk = b_per_w // BLK
    n_step = HIST * n_blk
    mesh = plsc.VectorSubcoreMesh(
        core_axis_name="c", subcore_axis_name="s", num_cores=NC, num_subcores=NS)

    @functools.partial(
        pl.kernel,
        out_type=jax.ShapeDtypeStruct((HIST, D, BATCH), jnp.float32),
        mesh=mesh,
        scratch_types=[
            pltpu.VMEM((HIST, 4 * BLK), jnp.int32),
            pltpu.VMEM((BLK,), jnp.int32),
            pltpu.VMEM((BLK,), jnp.int32),
            pltpu.VMEM((BLK,), jnp.int32),
            pltpu.VMEM((BLK,), jnp.int32),
            pltpu.VMEM((BLK, 128), jnp.float32),
            pltpu.VMEM((BLK, 128), jnp.float32),
            pltpu.VMEM((D, BLK), jnp.float32),
            pltpu.VMEM((D, BLK), jnp.float32),
            pltpu.SemaphoreType.DMA,
            pltpu.SemaphoreType.DMA,
            pltpu.SemaphoreType.DMA,
            pltpu.SemaphoreType.DMA,
            pltpu.SemaphoreType.DMA,
        ],
        compiler_params=pltpu.CompilerParams(needs_layout_passes=False),
    )
    def gather_kernel(idx_hbm, packed_hbm, out_hbm,
                      slab, gia, gib, qa, qb, ra, rb, oa, ob,
                      ssem, gsa, gsb, osa, osb):
        wid = lax.axis_index("s") * NC + lax.axis_index("c")
        base = wid * b_per_w
        iota = lax.iota(jnp.int32, LANES)

        def build(t, gi, q):
            h = t >> 2
            j = t & 3
            for k in range(BLK // LANES):
                v = slab[h, pl.ds(j * BLK + k * LANES, LANES)]
                gi[pl.ds(k * LANES, LANES)] = lax.shift_right_logical(v, 2)
                q[pl.ds(k * LANES, LANES)] = lax.shift_left(v & 3, 5)

        def fire_gather(gi, r, sem):
            return pltpu.async_copy(packed_hbm.at[gi], r, sem)

        def wait_gather(gi, r, sem):
            pltpu.make_async_copy(packed_hbm.at[gi], r, sem).wait()

        def compact(r, q, o):
            for b0 in range(BLK // LANES):
                rows = b0 * LANES + iota
                qv = q[pl.ds(b0 * LANES, LANES)]
                for d0 in range(D):
                    dvec = (d0 + iota) & (D - 1)
                    vals = plsc.load_gather(r, [rows, qv + dvec])
                    plsc.store_scatter(o, [dvec, rows], vals)

        def store(o, t, sem):
            h = t >> 2
            j = t & 3
            return pltpu.async_copy(
                o, out_hbm.at[h, :, pl.ds(base + j * BLK, BLK)], sem)

        def wait_store(o, sem):
            pltpu.make_async_copy(o, out_hbm.at[0, :, pl.ds(0, BLK)], sem).wait()

        def stage(h):
            pltpu.async_copy(
                idx_hbm.at[pl.ds(h * BATCH + base, n_blk * BLK)],
                slab.at[h], ssem)
        pl.loop(0, HIST)(stage)

        def drain(h):
            pltpu.make_async_copy(
                idx_hbm.at[pl.ds(0, n_blk * BLK)], slab.at[0], ssem).wait()
        pl.loop(0, HIST)(drain)

        build(0, gia, qa)
        fire_gather(gia, ra, gsa)

        def body(s):
            t0 = 2 * s

            build(t0 + 1, gib, qb)
            fire_gather(gib, rb, gsb)

            @pl.when(s > 0)
            def _():
                wait_store(oa, osa)
            wait_gather(gia, ra, gsa)
            compact(ra, qa, oa)
            store(oa, t0, osa)

            @pl.when(t0 + 2 < n_step)
            def _():
                build(t0 + 2, gia, qa)
                fire_gather(gia, ra, gsa)

            @pl.when(s > 0)
            def _():
                wait_store(ob, osb)
            wait_gather(gib, rb, gsb)
            compact(rb, qb, ob)
            store(ob, t0 + 1, osb)

        pl.loop(0, n_step // 2)(body)
        wait_store(oa, osa)
        wait_store(ob, osb)

    return gather_kernel


def kernel(inputs, table):
    BATCH, HIST = inputs.shape
    V, D = table.shape
    packed = jnp.reshape(table, (V * D // 128, 128))
    idx1d = jnp.reshape(jnp.transpose(inputs.astype(jnp.int32)), (BATCH * HIST,))
    outT = _make_gather(BATCH, HIST, D, V * D // 128)(idx1d, packed)
    return jnp.transpose(outT, (2, 0, 1))
